# baseline (speedup 1.0000x reference)
import functools

import jax
import jax.numpy as jnp
from jax import lax
from jax.experimental import pallas as pl
from jax.experimental.pallas import tpu as pltpu
from jax.experimental.pallas import tpu_sc as plsc

NUM_E = 1000000
DIM = 32
WORDS = DIM // 4
BATCH = 16384

_info = plsc.get_sparse_core_info()
NC, NS, LANES = _info.num_cores, _info.num_subcores, _info.num_lanes
NW = NC * NS
BPW = BATCH // NW
CHUNK = 128
NCHUNK = BPW // CHUNK


def _body(qw_hbm, idx_hbm, s_hbm, zp_hbm, out_hbm,
          idx_v, rows_v, s_v, zp_v, out_v, sem):
    c = lax.axis_index("c")
    s = lax.axis_index("s")
    wid = s * NC + c
    base = wid * BPW

    for j in range(NCHUNK):
        pltpu.sync_copy(idx_hbm.at[pl.ds(base + j * CHUNK, CHUNK)],
                        idx_v.at[j])

    copies = []
    for j in range(NCHUNK):
        copies.append(pltpu.async_copy(
            qw_hbm.at[idx_v.at[j]], rows_v.at[pl.ds(j * CHUNK, CHUNK)], sem))
        copies.append(pltpu.async_copy(
            s_hbm.at[idx_v.at[j]], s_v.at[pl.ds(j * CHUNK, CHUNK)], sem))
        copies.append(pltpu.async_copy(
            zp_hbm.at[idx_v.at[j]], zp_v.at[pl.ds(j * CHUNK, CHUNK)], sem))
    for cp in copies:
        cp.wait()

    iot = lax.iota(jnp.int32, LANES)

    def group(g, carry):
        r0 = g * LANES
        rvi = r0 + iot
        sv = s_v[pl.ds(r0, LANES)]
        zv = zp_v[pl.ds(r0, LANES)]
        for w in range(WORDS):
            wd = plsc.load_gather(
                rows_v, [rvi, jnp.full((LANES,), w, jnp.int32)])
            for b in range(4):
                if b == 0:
                    byte = wd & 255
                elif b == 3:
                    byte = lax.shift_right_logical(wd, 24)
                else:
                    byte = lax.shift_right_logical(wd, 8 * b) & 255
                f = (byte.astype(jnp.float32) - zv) * sv
                plsc.store_scatter(
                    out_v, [rvi, jnp.full((LANES,), 4 * w + b, jnp.int32)], f)
        return carry

    lax.fori_loop(0, BPW // LANES, group, 0)
    pltpu.sync_copy(out_v, out_hbm.at[pl.ds(base, BPW)])


def _run(qw32, idx, scales, zps):
    mesh = plsc.VectorSubcoreMesh(core_axis_name="c", subcore_axis_name="s")
    k = functools.partial(
        pl.kernel,
        out_type=jax.ShapeDtypeStruct((BATCH, DIM), jnp.float32),
        mesh=mesh,
        scratch_types=[
            pltpu.VMEM((NCHUNK, CHUNK), jnp.int32),
            pltpu.VMEM((BPW, WORDS), jnp.int32),
            pltpu.VMEM((BPW,), jnp.float32),
            pltpu.VMEM((BPW,), jnp.float32),
            pltpu.VMEM((BPW, DIM), jnp.float32),
            pltpu.SemaphoreType.DMA,
        ],
        compiler_params=pltpu.CompilerParams(
            needs_layout_passes=False, use_tc_tiling_on_sc=False),
    )(_body)
    return k(qw32, idx, scales, zps)


def kernel(indices, qweight, scales, zero_points):
    qw32 = lax.bitcast_convert_type(
        qweight.reshape(NUM_E, WORDS, 4), jnp.int32)
    idx = indices.astype(jnp.int32)
    return _run(qw32, idx, scales, zero_points)
